# baseline (speedup 1.0000x reference)
import jax
import jax.numpy as jnp
from jax import lax
from jax.experimental import pallas as pl
from jax.experimental.pallas import tpu as pltpu
from jax.experimental.pallas import tpu_sc as plsc

EMBEDDING_DIM = 64

NC = 2
NS = 16
NW = NC * NS

HALF = 100
NBUF = 8


def _gather_body(idx_hbm, table_hbm, out_hbm, idx_v, rows_v, gsem, osem):
    wid = lax.axis_index("s") * NC + lax.axis_index("c")
    n_chunks = idx_v.shape[0]
    rows_per_w = n_chunks // 2
    pltpu.sync_copy(idx_hbm.at[pl.ds(wid * n_chunks, n_chunks)], idx_v)
    row_base = wid * rows_per_w

    def start_gather(g, b):
        pltpu.make_async_copy(
            table_hbm.at[idx_v.at[g]], rows_v.at[b], gsem.at[b]).start()

    def wait_gather(g, b):
        pltpu.make_async_copy(
            table_hbm.at[idx_v.at[g]], rows_v.at[b], gsem.at[b]).wait()

    def out_copy(g, b):
        dst = out_hbm.at[row_base + g // 2, pl.ds((g % 2) * HALF, HALF)]
        return pltpu.make_async_copy(rows_v.at[b], dst, osem.at[b])

    for b in range(NBUF):
        start_gather(b, b)

    @pl.loop(0, n_chunks - NBUF, step=NBUF)
    def _(i):
        for b in range(NBUF):
            g = i + b
            wait_gather(g, b)
            out_copy(g, b).start()
            out_copy(g, b).wait()
            start_gather(g + NBUF, b)

    for b in range(NBUF):
        g = n_chunks - NBUF + b
        wait_gather(g, b)
        out_copy(g, b).start()
    for b in range(NBUF):
        g = n_chunks - NBUF + b
        out_copy(g, b).wait()


def kernel(antenna_indices, embeddings):
    batch, num_antennas = antenna_indices.shape
    total = batch * num_antennas
    assert num_antennas == 2 * HALF and batch % NW == 0

    idx2d = antenna_indices.astype(jnp.int32).reshape(total // HALF, HALF)

    mesh = plsc.VectorSubcoreMesh(core_axis_name="c", subcore_axis_name="s")
    run = pl.kernel(
        _gather_body,
        out_type=jax.ShapeDtypeStruct(
            (batch, num_antennas, EMBEDDING_DIM), jnp.float32),
        mesh=mesh,
        scratch_types=[
            pltpu.VMEM((total // HALF // NW, HALF), jnp.int32),
            pltpu.VMEM((NBUF, HALF, EMBEDDING_DIM), jnp.float32),
            pltpu.SemaphoreType.DMA((NBUF,)),
            pltpu.SemaphoreType.DMA((NBUF,)),
        ],
        compiler_params=pltpu.CompilerParams(use_tc_tiling_on_sc=False),
    )
    return run(idx2d, embeddings)


# speedup vs baseline: 1.3133x; 1.3133x over previous
import jax
import jax.numpy as jnp
from jax import lax
from jax.experimental import pallas as pl
from jax.experimental.pallas import tpu as pltpu
from jax.experimental.pallas import tpu_sc as plsc

EMBEDDING_DIM = 64
PADDED_DIM = 128

NC = 2
NS = 16
NW = NC * NS

CHUNK = 128
NBUF = 4


def _gather_body(idx_hbm, table_hbm, out_hbm, idx_v, rows_v, gsem, osem):
    wid = lax.axis_index("s") * NC + lax.axis_index("c")
    n_chunks = idx_v.shape[0]
    pltpu.sync_copy(idx_hbm.at[pl.ds(wid * n_chunks, n_chunks)], idx_v)
    row_base = wid * n_chunks * CHUNK

    def start_gather(g, b):
        pltpu.make_async_copy(
            table_hbm.at[idx_v.at[g]], rows_v.at[b], gsem.at[b]).start()

    def wait_gather(g, b):
        pltpu.make_async_copy(
            table_hbm.at[idx_v.at[g]], rows_v.at[b], gsem.at[b]).wait()

    def out_copy(g, b):
        return pltpu.make_async_copy(
            rows_v.at[b], out_hbm.at[pl.ds(row_base + g * CHUNK, CHUNK)],
            osem.at[b])

    for b in range(NBUF):
        start_gather(b, b)

    @pl.loop(0, n_chunks - NBUF, step=NBUF)
    def _(i):
        for b in range(NBUF):
            g = i + b
            wait_gather(g, b)
            out_copy(g, b).start()
            out_copy(g, b).wait()
            start_gather(g + NBUF, b)

    for b in range(NBUF):
        g = n_chunks - NBUF + b
        wait_gather(g, b)
        out_copy(g, b).start()
    for b in range(NBUF):
        g = n_chunks - NBUF + b
        out_copy(g, b).wait()


def kernel(antenna_indices, embeddings):
    batch, num_antennas = antenna_indices.shape
    total = batch * num_antennas
    assert total % (NW * CHUNK) == 0
    n_chunks = total // (NW * CHUNK)

    idx2d = antenna_indices.astype(jnp.int32).reshape(total // CHUNK, CHUNK)
    table = jnp.pad(embeddings, ((0, 0), (0, PADDED_DIM - EMBEDDING_DIM)))

    mesh = plsc.VectorSubcoreMesh(core_axis_name="c", subcore_axis_name="s")
    run = pl.kernel(
        _gather_body,
        out_type=jax.ShapeDtypeStruct((total, PADDED_DIM), jnp.float32),
        mesh=mesh,
        scratch_types=[
            pltpu.VMEM((n_chunks, CHUNK), jnp.int32),
            pltpu.VMEM((NBUF, CHUNK, PADDED_DIM), jnp.float32),
            pltpu.SemaphoreType.DMA((NBUF,)),
            pltpu.SemaphoreType.DMA((NBUF,)),
        ],
        compiler_params=pltpu.CompilerParams(use_tc_tiling_on_sc=False),
    )
    out = run(idx2d, table)
    return out[:, :EMBEDDING_DIM].reshape(batch, num_antennas, EMBEDDING_DIM)


# speedup vs baseline: 1.7681x; 1.3463x over previous
import jax
import jax.numpy as jnp
from jax import lax
from jax.experimental import pallas as pl
from jax.experimental.pallas import tpu as pltpu
from jax.experimental.pallas import tpu_sc as plsc

EMBEDDING_DIM = 64
PADDED_DIM = 128

NC = 2
NS = 16
NW = NC * NS

CHUNK = 128
NBUF = 4


def _gather_body(idx_hbm, table_hbm, out_hbm, idx_v, rows_v, gsem, osem):
    wid = lax.axis_index("s") * NC + lax.axis_index("c")
    n_chunks = idx_v.shape[0]
    pltpu.sync_copy(idx_hbm.at[pl.ds(wid * n_chunks, n_chunks)], idx_v)
    row_base = wid * n_chunks * CHUNK

    def start_gather(g, b):
        pltpu.make_async_copy(
            table_hbm.at[idx_v.at[g]], rows_v.at[b], gsem.at[b]).start()

    def wait_gather(g, b):
        pltpu.make_async_copy(
            table_hbm.at[idx_v.at[g]], rows_v.at[b], gsem.at[b]).wait()

    def out_copy(g, b):
        return pltpu.make_async_copy(
            rows_v.at[b],
            out_hbm.at[pl.ds(row_base + g * CHUNK, CHUNK),
                       pl.ds(0, EMBEDDING_DIM)],
            osem.at[b])

    for b in range(NBUF):
        start_gather(b, b)

    @pl.loop(0, n_chunks - NBUF, step=NBUF)
    def _(i):
        for b in range(NBUF):
            g = i + b
            wait_gather(g, b)
            out_copy(g, b).start()
            out_copy(g, b).wait()
            start_gather(g + NBUF, b)

    for b in range(NBUF):
        g = n_chunks - NBUF + b
        wait_gather(g, b)
        out_copy(g, b).start()
    for b in range(NBUF):
        g = n_chunks - NBUF + b
        out_copy(g, b).wait()


def kernel(antenna_indices, embeddings):
    batch, num_antennas = antenna_indices.shape
    total = batch * num_antennas
    assert total % (NW * CHUNK) == 0
    n_chunks = total // (NW * CHUNK)

    idx2d = antenna_indices.astype(jnp.int32).reshape(total // CHUNK, CHUNK)

    mesh = plsc.VectorSubcoreMesh(core_axis_name="c", subcore_axis_name="s")
    run = pl.kernel(
        _gather_body,
        out_type=jax.ShapeDtypeStruct((total, PADDED_DIM), jnp.float32),
        mesh=mesh,
        scratch_types=[
            pltpu.VMEM((n_chunks, CHUNK), jnp.int32),
            pltpu.VMEM((NBUF, CHUNK, EMBEDDING_DIM), jnp.float32),
            pltpu.SemaphoreType.DMA((NBUF,)),
            pltpu.SemaphoreType.DMA((NBUF,)),
        ],
        compiler_params=pltpu.CompilerParams(use_tc_tiling_on_sc=False),
    )
    out = run(idx2d, embeddings)
    return out[:, :EMBEDDING_DIM].reshape(batch, num_antennas, EMBEDDING_DIM)
